# Initial kernel scaffold; baseline (speedup 1.0000x reference)
import jax
import jax.numpy as jnp
from jax.experimental import pallas as pl


def kernel(x, wk, bc, w1, b1, w2, b2):
    raise NotImplementedError("write your pallas kernel here")


# speedup vs baseline: 1.6794x; 1.6794x over previous
import functools

import jax
import jax.numpy as jnp
from jax.experimental import pallas as pl
from jax.experimental.pallas import tpu as pltpu

_B = 8


def _conv_pool_kernel(h, w, x_ref, pmask_ref, wk_ref, bc_ref, pooled_ref):
    wp2 = w + 2
    rows = h * wp2
    span = rows + 2 * wp2

    convs = []
    for b in range(_B):
        win = x_ref[b]
        wincat = jnp.concatenate([win[dj:dj + span, :] for dj in range(3)],
                                 axis=-1)
        patches = jnp.concatenate(
            [wincat[di * wp2:di * wp2 + rows, :] for di in range(3)],
            axis=-1)
        conv = jnp.dot(patches, wk_ref[...],
                       preferred_element_type=jnp.float32)
        convs.append(jnp.maximum(conv + bc_ref[...], 0.0))

    convcat = jnp.concatenate(convs, axis=0)
    pooled_ref[...] = jnp.dot(pmask_ref[...], convcat,
                              preferred_element_type=jnp.float32)


def _head_kernel(p_ref, w1_ref, b1_ref, w2_ref, b2_ref, logits_ref, feat_ref):
    emb = jnp.maximum(
        jnp.dot(p_ref[...].astype(jnp.bfloat16), w1_ref[...],
                preferred_element_type=jnp.float32) + b1_ref[...], 0.0)
    feat_ref[...] = emb
    logits_ref[...] = (jnp.dot(emb.astype(jnp.bfloat16), w2_ref[...],
                               preferred_element_type=jnp.float32)
                       + b2_ref[...])


@jax.jit
def _forward(x, wk, bc, w1, b1, w2, b2):
    n, c, h, w = x.shape
    wp2 = w + 2
    rows = h * wp2
    rows_pad = (h + 4) * wp2

    fpad = wk.shape[-1]
    epad = w1.shape[-1]
    cpad = w2.shape[-1]

    x_nhwc = jnp.transpose(x, (0, 2, 3, 1))
    x_pad = jnp.pad(x_nhwc, ((0, 0), (1, 3), (1, 1), (0, 0))).astype(jnp.bfloat16)
    x_flat = x_pad.reshape(n, rows_pad, c)

    base = jnp.where(jnp.arange(rows) % wp2 < w, 1.0 / (h * w), 0.0)
    pmask = jnp.kron(jnp.eye(_B, dtype=jnp.float32),
                     base.astype(jnp.float32).reshape(1, rows))

    pooled = pl.pallas_call(
        functools.partial(_conv_pool_kernel, h, w),
        out_shape=jax.ShapeDtypeStruct((n, fpad), jnp.float32),
        grid=(n // _B,),
        in_specs=[
            pl.BlockSpec((_B, rows_pad, c), lambda i: (i, 0, 0)),
            pl.BlockSpec((_B, _B * rows), lambda i: (0, 0)),
            pl.BlockSpec((9 * c, fpad), lambda i: (0, 0)),
            pl.BlockSpec((1, fpad), lambda i: (0, 0)),
        ],
        out_specs=pl.BlockSpec((_B, fpad), lambda i: (i, 0)),
        compiler_params=pltpu.CompilerParams(
            dimension_semantics=("parallel",),
            vmem_limit_bytes=64 * 1024 * 1024,
        ),
    )(x_flat, pmask, wk, bc)

    bm = n // 2
    logits_pad, feat_pad = pl.pallas_call(
        _head_kernel,
        out_shape=(
            jax.ShapeDtypeStruct((n, cpad), jnp.float32),
            jax.ShapeDtypeStruct((n, epad), jnp.float32),
        ),
        grid=(2,),
        in_specs=[
            pl.BlockSpec((bm, fpad), lambda i: (i, 0)),
            pl.BlockSpec((fpad, epad), lambda i: (0, 0)),
            pl.BlockSpec((1, epad), lambda i: (0, 0)),
            pl.BlockSpec((epad, cpad), lambda i: (0, 0)),
            pl.BlockSpec((1, cpad), lambda i: (0, 0)),
        ],
        out_specs=(
            pl.BlockSpec((bm, cpad), lambda i: (i, 0)),
            pl.BlockSpec((bm, epad), lambda i: (i, 0)),
        ),
        compiler_params=pltpu.CompilerParams(
            dimension_semantics=("parallel",),
        ),
    )(pooled, w1, b1, w2, b2)

    return logits_pad[:, :1000], feat_pad[:, :256]


def kernel(x, wk, bc, w1, b1, w2, b2):
    return _forward(x, wk, bc, w1, b1, w2, b2)
